# baseline (speedup 1.0000x reference)
import functools

import numpy as np
import jax
import jax.numpy as jnp
from jax import lax
from jax.experimental import pallas as pl
from jax.experimental.pallas import tpu as pltpu
from jax.experimental.pallas import tpu_sc as plsc

NC = 2
NS = 16
NW = NC * NS
L = 16

B = 4
S = 2048
D = 2048

POS_PER_W = S // NW
CH = 16
N_CH = POS_PER_W // CH


def _positional_encoding(seq_len, d_model):
  pos = jnp.arange(seq_len, dtype=jnp.float32)[:, None]
  div = jnp.exp(
      jnp.arange(0, d_model, 2, dtype=jnp.float32)
      * (-np.log(10000.0) / d_model))
  ang = pos * div[None, :]
  pe = jnp.zeros((seq_len, d_model), dtype=jnp.float32)
  pe = pe.at[:, 0::2].set(jnp.sin(ang))
  pe = pe.at[:, 1::2].set(jnp.cos(ang))
  return pe


def _body(emb_hbm, idx_hbm, pe_hbm, out_hbm, idx_v, rows_v, pe_v, sem):
  wid = lax.axis_index("s") * NC + lax.axis_index("c")
  pos0 = wid * POS_PER_W
  for c in range(N_CH):
    pbase = pos0 + c * CH
    pltpu.sync_copy(pe_hbm.at[pl.ds(pbase, CH)], pe_v)
    for b in range(B):
      tok = b * S + pbase
      pltpu.sync_copy(idx_hbm.at[pl.ds(tok, CH)], idx_v)
      pltpu.async_copy(emb_hbm.at[idx_v], rows_v, sem).wait()

      def add_j(j, _):
        col = j * L
        for r in range(CH):
          rows_v[r, pl.ds(col, L)] = (
              rows_v[r, pl.ds(col, L)] + pe_v[r, pl.ds(col, L)])
        return 0

      lax.fori_loop(0, D // L, add_j, 0)
      pltpu.sync_copy(rows_v, out_hbm.at[pl.ds(tok, CH)])


@jax.jit
def _run(emb, idx, pe):
  mesh = plsc.VectorSubcoreMesh(
      core_axis_name="c", subcore_axis_name="s",
      num_cores=NC, num_subcores=NS)
  fn = functools.partial(
      pl.kernel,
      out_type=jax.ShapeDtypeStruct((B * S, D), jnp.float32),
      mesh=mesh,
      scratch_types=[
          pltpu.VMEM((CH,), jnp.int32),
          pltpu.VMEM((CH, D), jnp.float32),
          pltpu.VMEM((CH, D), jnp.float32),
          pltpu.SemaphoreType.DMA,
      ],
  )(_body)
  return fn(emb, idx, pe)


def kernel(x, emb):
  idx = x.reshape(-1).astype(jnp.int32)
  pe = _positional_encoding(S, D)
  out = _run(emb, idx, pe)
  return out.reshape(B, S, D)


# speedup vs baseline: 1.2474x; 1.2474x over previous
import functools

import numpy as np
import jax
import jax.numpy as jnp
from jax import lax
from jax.experimental import pallas as pl
from jax.experimental.pallas import tpu as pltpu
from jax.experimental.pallas import tpu_sc as plsc

NC = 2
NS = 16
NW = NC * NS
L = 16

B = 4
S = 2048
D = 2048

POS_PER_W = S // NW
CH = 8
N_CH = POS_PER_W // CH
N_U = N_CH * B
NB = 4


def _positional_encoding(seq_len, d_model):
  pos = jnp.arange(seq_len, dtype=jnp.float32)[:, None]
  div = jnp.exp(
      jnp.arange(0, d_model, 2, dtype=jnp.float32)
      * (-np.log(10000.0) / d_model))
  ang = pos * div[None, :]
  pe = jnp.zeros((seq_len, d_model), dtype=jnp.float32)
  pe = pe.at[:, 0::2].set(jnp.sin(ang))
  pe = pe.at[:, 1::2].set(jnp.cos(ang))
  return pe


def _body(emb_hbm, idx_hbm, pe_hbm, out_hbm,
          idx_all, r0, r1, r2, r3, pe0, pe1,
          sem_g, sem_st, sem_pe):
  rows = [r0, r1, r2, r3]
  pes = [pe0, pe1]
  wid = lax.axis_index("s") * NC + lax.axis_index("c")
  pos0 = wid * POS_PER_W

  units = [(c, b) for c in range(N_CH) for b in range(B)]

  for b in range(B):
    pltpu.sync_copy(idx_hbm.at[pl.ds(b * S + pos0, POS_PER_W)],
                    idx_all.at[b])
  pltpu.sync_copy(pe_hbm.at[pl.ds(pos0, CH)], pes[0])

  def start_gather(u):
    c, b = units[u]
    idx_ref = idx_all.at[b, pl.ds(c * CH, CH)]
    return pltpu.async_copy(emb_hbm.at[idx_ref], rows[u % NB], sem_g)

  gathers = {0: start_gather(0), 1: start_gather(1)}
  stores = {}
  pe_cps = {}

  for u in range(N_U):
    c, b = units[u]
    if b == 0 and c + 1 < N_CH:
      pe_cps[c + 1] = pltpu.async_copy(
          pe_hbm.at[pl.ds(pos0 + (c + 1) * CH, CH)],
          pes[(c + 1) % 2], sem_pe)
    if u + 2 < N_U:
      if u - 2 >= 0:
        stores[u - 2].wait()
      gathers[u + 2] = start_gather(u + 2)
    gathers[u].wait()
    if b == 0 and c > 0:
      pe_cps[c].wait()
    rv = rows[u % NB]
    pv = pes[c % 2]

    def add_j(j, _):
      col = j * L
      for r in range(CH):
        rv[r, pl.ds(col, L)] = rv[r, pl.ds(col, L)] + pv[r, pl.ds(col, L)]
      return 0

    lax.fori_loop(0, D // L, add_j, 0)
    tok = b * S + pos0 + c * CH
    stores[u] = pltpu.async_copy(rv, out_hbm.at[pl.ds(tok, CH)], sem_st)

  for u in range(N_U - 4, N_U):
    stores[u].wait()


@jax.jit
def _run(emb, idx, pe):
  mesh = plsc.VectorSubcoreMesh(
      core_axis_name="c", subcore_axis_name="s",
      num_cores=NC, num_subcores=NS)
  fn = functools.partial(
      pl.kernel,
      out_type=jax.ShapeDtypeStruct((B * S, D), jnp.float32),
      mesh=mesh,
      scratch_types=[
          pltpu.VMEM((B, POS_PER_W), jnp.int32),
          pltpu.VMEM((CH, D), jnp.float32),
          pltpu.VMEM((CH, D), jnp.float32),
          pltpu.VMEM((CH, D), jnp.float32),
          pltpu.VMEM((CH, D), jnp.float32),
          pltpu.VMEM((CH, D), jnp.float32),
          pltpu.VMEM((CH, D), jnp.float32),
          pltpu.SemaphoreType.DMA,
          pltpu.SemaphoreType.DMA,
          pltpu.SemaphoreType.DMA,
      ],
  )(_body)
  return fn(emb, idx, pe)


def kernel(x, emb):
  idx = x.reshape(-1).astype(jnp.int32)
  pe = _positional_encoding(S, D)
  out = _run(emb, idx, pe)
  return out.reshape(B, S, D)


# speedup vs baseline: 1.3622x; 1.0920x over previous
import functools

import numpy as np
import jax
import jax.numpy as jnp
from jax import lax
from jax.experimental import pallas as pl
from jax.experimental.pallas import tpu as pltpu
from jax.experimental.pallas import tpu_sc as plsc

NC = 2
NS = 16
NW = NC * NS
L = 16

B = 4
S = 2048
D = 2048

POS_PER_W = S // NW
CH = 8
N_CH = POS_PER_W // CH
N_U = N_CH * B
NB = 5


def _positional_encoding(seq_len, d_model):
  pos = jnp.arange(seq_len, dtype=jnp.float32)[:, None]
  div = jnp.exp(
      jnp.arange(0, d_model, 2, dtype=jnp.float32)
      * (-np.log(10000.0) / d_model))
  ang = pos * div[None, :]
  pe = jnp.zeros((seq_len, d_model), dtype=jnp.float32)
  pe = pe.at[:, 0::2].set(jnp.sin(ang))
  pe = pe.at[:, 1::2].set(jnp.cos(ang))
  return pe


def _body(emb_hbm, idx_hbm, pe_hbm, out_hbm,
          idx_all, r0, r1, r2, r3, r4, pe0, pe1,
          sem_g, sem_st, sem_pe):
  rows = [r0, r1, r2, r3, r4]
  pes = [pe0, pe1]
  wid = lax.axis_index("s") * NC + lax.axis_index("c")
  pos0 = wid * POS_PER_W

  units = [(c, b) for c in range(N_CH) for b in range(B)]

  for b in range(B):
    pltpu.sync_copy(idx_hbm.at[pl.ds(b * S + pos0, POS_PER_W)],
                    idx_all.at[b])
  pltpu.sync_copy(pe_hbm.at[pl.ds(pos0, CH)], pes[0])

  def start_gather(u):
    c, b = units[u]
    idx_ref = idx_all.at[b, pl.ds(c * CH, CH)]
    return pltpu.async_copy(emb_hbm.at[idx_ref], rows[u % NB], sem_g)

  gathers = {u: start_gather(u) for u in range(3)}
  stores = {}
  pe_cps = {}

  for u in range(N_U):
    c, b = units[u]
    if b == 0 and c + 1 < N_CH:
      pe_cps[c + 1] = pltpu.async_copy(
          pe_hbm.at[pl.ds(pos0 + (c + 1) * CH, CH)],
          pes[(c + 1) % 2], sem_pe)
    if u + 3 < N_U:
      if u - 2 >= 0:
        stores[u - 2].wait()
      gathers[u + 3] = start_gather(u + 3)
    gathers[u].wait()
    if b == 0 and c > 0:
      pe_cps[c].wait()
    rv = rows[u % NB]
    pv = pes[c % 2]

    @plsc.parallel_loop(0, D // L, 1, unroll=4)
    def add_j(j):
      col = j * L
      for r in range(CH):
        rv[r, pl.ds(col, L)] = rv[r, pl.ds(col, L)] + pv[r, pl.ds(col, L)]

    tok = b * S + pos0 + c * CH
    stores[u] = pltpu.async_copy(rv, out_hbm.at[pl.ds(tok, CH)], sem_st)

  for u in range(N_U - 5, N_U):
    stores[u].wait()


@jax.jit
def _run(emb, idx, pe):
  mesh = plsc.VectorSubcoreMesh(
      core_axis_name="c", subcore_axis_name="s",
      num_cores=NC, num_subcores=NS)
  fn = functools.partial(
      pl.kernel,
      out_type=jax.ShapeDtypeStruct((B * S, D), jnp.float32),
      mesh=mesh,
      scratch_types=[
          pltpu.VMEM((B, POS_PER_W), jnp.int32),
          pltpu.VMEM((CH, D), jnp.float32),
          pltpu.VMEM((CH, D), jnp.float32),
          pltpu.VMEM((CH, D), jnp.float32),
          pltpu.VMEM((CH, D), jnp.float32),
          pltpu.VMEM((CH, D), jnp.float32),
          pltpu.VMEM((CH, D), jnp.float32),
          pltpu.VMEM((CH, D), jnp.float32),
          pltpu.SemaphoreType.DMA,
          pltpu.SemaphoreType.DMA,
          pltpu.SemaphoreType.DMA,
      ],
  )(_body)
  return fn(emb, idx, pe)


def kernel(x, emb):
  idx = x.reshape(-1).astype(jnp.int32)
  pe = _positional_encoding(S, D)
  out = _run(emb, idx, pe)
  return out.reshape(B, S, D)


# speedup vs baseline: 2.8056x; 2.0597x over previous
import functools

import numpy as np
import jax
import jax.numpy as jnp
from jax import lax
from jax.experimental import pallas as pl
from jax.experimental.pallas import tpu as pltpu
from jax.experimental.pallas import tpu_sc as plsc

NC = 2
NS = 16
NW = NC * NS
L = 16

B = 4
S = 2048
D = 2048

POS_PER_W = S // NW
CH = 8
N_CH = POS_PER_W // CH
N_U = N_CH * B
NB = 5


def _positional_encoding(seq_len, d_model):
  pos = np.arange(seq_len, dtype=np.float32)[:, None]
  div = np.exp(
      np.arange(0, d_model, 2, dtype=np.float32)
      * (-np.log(10000.0) / d_model))
  ang = (pos * div[None, :]).astype(np.float32)
  pe = np.zeros((seq_len, d_model), dtype=np.float32)
  pe[:, 0::2] = np.sin(ang)
  pe[:, 1::2] = np.cos(ang)
  return jnp.asarray(pe)


def _body(emb_hbm, idx_hbm, pe_hbm, out_hbm,
          idx_all, r0, r1, r2, r3, r4, pe0, pe1,
          sem_g, sem_st, sem_pe):
  rows = [r0, r1, r2, r3, r4]
  pes = [pe0, pe1]
  wid = lax.axis_index("s") * NC + lax.axis_index("c")
  pos0 = wid * POS_PER_W

  units = [(c, b) for c in range(N_CH) for b in range(B)]

  for b in range(B):
    pltpu.sync_copy(idx_hbm.at[pl.ds(b * S + pos0, POS_PER_W)],
                    idx_all.at[b])
  pltpu.sync_copy(pe_hbm.at[pl.ds(pos0, CH)], pes[0])

  def start_gather(u):
    c, b = units[u]
    idx_ref = idx_all.at[b, pl.ds(c * CH, CH)]
    return pltpu.async_copy(emb_hbm.at[idx_ref], rows[u % NB], sem_g)

  gathers = {u: start_gather(u) for u in range(3)}
  stores = {}
  pe_cps = {}

  for u in range(N_U):
    c, b = units[u]
    if b == 0 and c + 1 < N_CH:
      pe_cps[c + 1] = pltpu.async_copy(
          pe_hbm.at[pl.ds(pos0 + (c + 1) * CH, CH)],
          pes[(c + 1) % 2], sem_pe)
    if u + 3 < N_U:
      if u - 2 >= 0:
        stores[u - 2].wait()
      gathers[u + 3] = start_gather(u + 3)
    gathers[u].wait()
    if b == 0 and c > 0:
      pe_cps[c].wait()
    rv = rows[u % NB]
    pv = pes[c % 2]

    @plsc.parallel_loop(0, D // L, 1, unroll=4)
    def add_j(j):
      col = j * L
      for r in range(CH):
        rv[r, pl.ds(col, L)] = rv[r, pl.ds(col, L)] + pv[r, pl.ds(col, L)]

    tok = b * S + pos0 + c * CH
    stores[u] = pltpu.async_copy(rv, out_hbm.at[pl.ds(tok, CH)], sem_st)

  for u in range(N_U - 5, N_U):
    stores[u].wait()


@jax.jit
def _run(emb, idx, pe):
  mesh = plsc.VectorSubcoreMesh(
      core_axis_name="c", subcore_axis_name="s",
      num_cores=NC, num_subcores=NS)
  fn = functools.partial(
      pl.kernel,
      out_type=jax.ShapeDtypeStruct((B * S, D), jnp.float32),
      mesh=mesh,
      scratch_types=[
          pltpu.VMEM((B, POS_PER_W), jnp.int32),
          pltpu.VMEM((CH, D), jnp.float32),
          pltpu.VMEM((CH, D), jnp.float32),
          pltpu.VMEM((CH, D), jnp.float32),
          pltpu.VMEM((CH, D), jnp.float32),
          pltpu.VMEM((CH, D), jnp.float32),
          pltpu.VMEM((CH, D), jnp.float32),
          pltpu.VMEM((CH, D), jnp.float32),
          pltpu.SemaphoreType.DMA,
          pltpu.SemaphoreType.DMA,
          pltpu.SemaphoreType.DMA,
      ],
  )(_body)
  return fn(emb, idx, pe)


def kernel(x, emb):
  idx = x.reshape(-1).astype(jnp.int32)
  pe = _positional_encoding(S, D)
  out = _run(emb, idx, pe)
  return out.reshape(B, S, D)
